# Initial kernel scaffold; baseline (speedup 1.0000x reference)
import jax
import jax.numpy as jnp
from jax.experimental import pallas as pl


def kernel(x, ner, pos, entity_table):
    raise NotImplementedError("write your pallas kernel here")


# speedup vs baseline: 1.9421x; 1.9421x over previous
import functools

import jax
import jax.numpy as jnp
from jax import lax
from jax.experimental import pallas as pl
from jax.experimental.pallas import tpu as pltpu
from jax.experimental.pallas import tpu_sc as plsc

_B, _S, _D = 1024, 200, 128
_E = 32
_N = _B * _S


def _sc_concat_embed(x2d, ner1d, table):
    info = plsc.get_sparse_core_info()
    nw = info.num_cores * info.num_subcores
    n_per_w = _N // nw
    chunk = 640
    steps = n_per_w // chunk

    mesh = plsc.VectorSubcoreMesh(core_axis_name="c", subcore_axis_name="s")

    @functools.partial(
        pl.kernel,
        mesh=mesh,
        out_type=jax.ShapeDtypeStruct((_N, _D + _E), jnp.float32),
        compiler_params=pltpu.CompilerParams(use_tc_tiling_on_sc=False),
        scratch_types=[
            pltpu.VMEM((chunk,), jnp.int32),
            pltpu.VMEM((chunk, _E), jnp.float32),
            pltpu.VMEM((chunk, _D), jnp.float32),
            pltpu.SemaphoreType.DMA,
        ],
    )
    def k(x_hbm, ner_hbm, table_hbm, out_hbm, idx_v, rows_v, x_v, sem):
        wid = lax.axis_index("s") * info.num_cores + lax.axis_index("c")
        base = wid * n_per_w

        def body(i, carry):
            off = base + i * chunk
            pltpu.sync_copy(ner_hbm.at[pl.ds(off, chunk)], idx_v)
            pltpu.sync_copy(x_hbm.at[pl.ds(off, chunk), :], x_v)
            gather = pltpu.async_copy(table_hbm.at[idx_v], rows_v, sem)
            pltpu.sync_copy(x_v, out_hbm.at[pl.ds(off, chunk), pl.ds(0, _D)])
            gather.wait()
            pltpu.sync_copy(rows_v, out_hbm.at[pl.ds(off, chunk), pl.ds(_D, _E)])
            return carry

        lax.fori_loop(0, steps, body, 0)

    return k(x2d, ner1d, table)


def kernel(x, ner, pos, entity_table):
    del pos
    x2d = x.reshape(_N, _D)
    ner1d = ner.reshape(_N).astype(jnp.int32)
    out = _sc_concat_embed(x2d, ner1d, entity_table)
    return out.reshape(_B, _S, _D + _E)
